# baseline (speedup 1.0000x reference)
import functools

import jax
import jax.numpy as jnp
from jax import lax
from jax.experimental import pallas as pl
from jax.experimental.pallas import tpu as pltpu
from jax.experimental.pallas import tpu_sc as plsc

B = 1024
L = 200
D = 64

NC = 2
NS = 16
NW = NC * NS

ROWS = B * L
RPW = ROWS // NW
C = 128
NCHUNK = RPW // C


def _collate_kernel(idx0, idx1, idx2, len0, len1, len2, table,
                    out0, out1, out2,
                    idx_v, rows_v, len_v, sem):
    wid = lax.axis_index("s") * NC + lax.axis_index("c")
    base0 = wid * RPW

    lanes = lax.iota(jnp.int32, 16)
    zeros = jnp.zeros((16,), jnp.float32)

    for idx_hbm, len_hbm, out_hbm in ((idx0, len0, out0),
                                      (idx1, len1, out1),
                                      (idx2, len2, out2)):
        pltpu.sync_copy(len_hbm, len_v)

        def chunk_body(ci, _, idx_hbm=idx_hbm, out_hbm=out_hbm):
            base = base0 + ci * C
            pltpu.sync_copy(idx_hbm.at[pl.ds(base, C)], idx_v)
            pltpu.async_copy(table.at[idx_v], rows_v, sem).wait()

            def group_body(g, _):
                ridx = g * 16 + lanes
                pvec = base + ridx
                b = pvec // L
                j = pvec - b * L
                lens = plsc.load_gather(len_v, [b])
                invalid = j >= lens

                @pl.when(jnp.any(invalid))
                def _():
                    for col in range(D):
                        plsc.store_scatter(
                            rows_v,
                            [ridx, jnp.full((16,), col, jnp.int32)],
                            zeros, mask=invalid)
                return 0

            lax.fori_loop(0, C // 16, group_body, 0)
            pltpu.sync_copy(rows_v, out_hbm.at[pl.ds(base, C)])
            return 0

        lax.fori_loop(0, NCHUNK, chunk_body, 0)


@jax.jit
def _collate(idx0, idx1, idx2, len0, len1, len2, table):
    mesh = plsc.VectorSubcoreMesh(core_axis_name="c", subcore_axis_name="s")
    f = functools.partial(
        pl.kernel,
        mesh=mesh,
        compiler_params=pltpu.CompilerParams(use_tc_tiling_on_sc=False,
                                             needs_layout_passes=False),
        out_type=[jax.ShapeDtypeStruct((ROWS, D), jnp.float32)] * 3,
        scratch_types=[
            pltpu.VMEM((C,), jnp.int32),
            pltpu.VMEM((C, D), jnp.float32),
            pltpu.VMEM((B,), jnp.int32),
            pltpu.SemaphoreType.DMA,
        ],
    )(_collate_kernel)
    return f(idx0, idx1, idx2, len0, len1, len2, table)


def kernel(trajs_idx, trajs1_idx, trajs2_idx, trajs_len, trajs1_len,
           trajs2_len, road_emb):
    i0 = trajs1_idx.reshape(-1).astype(jnp.int32)
    i1 = trajs2_idx.reshape(-1).astype(jnp.int32)
    i2 = trajs_idx.reshape(-1).astype(jnp.int32)
    l0 = trajs1_len.astype(jnp.int32)
    l1 = trajs2_len.astype(jnp.int32)
    l2 = trajs_len.astype(jnp.int32)
    o0, o1, o2 = _collate(i0, i1, i2, l0, l1, l2, road_emb)
    return (o0.reshape(B, L, D), trajs1_len,
            o1.reshape(B, L, D), trajs2_len,
            o2.reshape(B, L, D), trajs_len)
